# Initial kernel scaffold; baseline (speedup 1.0000x reference)
import jax
import jax.numpy as jnp
from jax.experimental import pallas as pl


def kernel(x, adj, loop_att, We1, We2, Wh1, Wh2, Wd, Wmlp, bmlp):
    raise NotImplementedError("write your pallas kernel here")


# speedup vs baseline: 1.1606x; 1.1606x over previous
import jax
import jax.numpy as jnp
from jax.experimental import pallas as pl

N = 4096
D = 128
BR = 256
NBLK = N // BR


def _deg_proj_kernel(adj_ref, x_ref, wenc_ref, la_ref, d_ref, dxw_ref):
    la = la_ref[0, 0]
    deg = jnp.sum(adj_ref[...], axis=1, keepdims=True) + la
    d = jnp.where(deg > 0, jax.lax.rsqrt(deg), 0.0)
    d_ref[...] = d
    xw = jnp.dot(x_ref[...], wenc_ref[...], preferred_element_type=jnp.float32)
    dxw_ref[...] = d * xw


def _layer1_kernel(adj_ref, v_ref, d_ref, la_ref, w2_ref, o_ref):
    i = pl.program_id(0)
    la = la_ref[0, 0]
    acc = jnp.dot(adj_ref[...], v_ref[...], preferred_element_type=jnp.float32)
    acc = acc + la * v_ref[pl.ds(i * BR, BR), :]
    h = jnp.maximum(d_ref[...] * acc, 0.0)
    o_ref[...] = d_ref[...] * jnp.dot(
        h, w2_ref[...], preferred_element_type=jnp.float32
    )


def _layer2_kernel(
    adj_ref, v_ref, d_ref, la_ref, wd_ref, wmlp_ref, bmlp_ref,
    e2_ref, hh2_ref, ln_ref, dz_ref,
):
    i = pl.program_id(0)
    la = la_ref[0, 0]
    acc = jnp.dot(adj_ref[...], v_ref[...], preferred_element_type=jnp.float32)
    acc = acc + la * v_ref[pl.ds(i * BR, BR), :]
    y = d_ref[...] * acc
    e2 = y[:, :D]
    hh2 = y[:, D:]
    e2_ref[...] = e2
    hh2_ref[...] = hh2
    ln_ref[...] = (
        jnp.dot(e2, wmlp_ref[...], preferred_element_type=jnp.float32)
        + bmlp_ref[...]
    )
    dz_ref[...] = d_ref[...] * jnp.dot(
        hh2, wd_ref[...], preferred_element_type=jnp.float32
    )


def _layer3_kernel(adj_ref, v_ref, d_ref, la_ref, o_ref):
    i = pl.program_id(0)
    la = la_ref[0, 0]
    acc = jnp.dot(adj_ref[...], v_ref[...], preferred_element_type=jnp.float32)
    acc = acc + la * v_ref[pl.ds(i * BR, BR), :]
    o_ref[...] = d_ref[...] * acc


def _row_block(i):
    return (i, 0)


def _full(i):
    return (0, 0)


def _adj_spec():
    return pl.BlockSpec((BR, N), _row_block)


@jax.jit
def kernel(x, adj, loop_att, We1, We2, Wh1, Wh2, Wd, Wmlp, bmlp):
    xs = x[0]
    A = adj[0]
    la = jnp.reshape(loop_att, (1, 1))

    wenc = jnp.concatenate([We1, Wh1], axis=1)
    zero = jnp.zeros((D, D), jnp.float32)
    w2 = jnp.concatenate(
        [
            jnp.concatenate([We2, zero], axis=1),
            jnp.concatenate([zero, Wh2], axis=1),
        ],
        axis=0,
    )
    wd_p = jnp.zeros((D, D), jnp.float32).at[:, : Wd.shape[1]].set(Wd)
    wmlp_p = jnp.zeros((D, D), jnp.float32).at[:, : Wmlp.shape[1]].set(Wmlp)
    bmlp_p = jnp.zeros((1, D), jnp.float32).at[0, : bmlp.shape[0]].set(bmlp)

    d, dxw = pl.pallas_call(
        _deg_proj_kernel,
        grid=(NBLK,),
        in_specs=[
            _adj_spec(),
            pl.BlockSpec((BR, D), _row_block),
            pl.BlockSpec((D, 2 * D), _full),
            pl.BlockSpec((1, 1), _full),
        ],
        out_specs=[
            pl.BlockSpec((BR, 1), _row_block),
            pl.BlockSpec((BR, 2 * D), _row_block),
        ],
        out_shape=[
            jax.ShapeDtypeStruct((N, 1), jnp.float32),
            jax.ShapeDtypeStruct((N, 2 * D), jnp.float32),
        ],
    )(A, xs, wenc, la)

    dhw = pl.pallas_call(
        _layer1_kernel,
        grid=(NBLK,),
        in_specs=[
            _adj_spec(),
            pl.BlockSpec((N, 2 * D), _full),
            pl.BlockSpec((BR, 1), _row_block),
            pl.BlockSpec((1, 1), _full),
            pl.BlockSpec((2 * D, 2 * D), _full),
        ],
        out_specs=pl.BlockSpec((BR, 2 * D), _row_block),
        out_shape=jax.ShapeDtypeStruct((N, 2 * D), jnp.float32),
    )(A, dxw, d, la, w2)

    e2, hh2, ln, dz = pl.pallas_call(
        _layer2_kernel,
        grid=(NBLK,),
        in_specs=[
            _adj_spec(),
            pl.BlockSpec((N, 2 * D), _full),
            pl.BlockSpec((BR, 1), _row_block),
            pl.BlockSpec((1, 1), _full),
            pl.BlockSpec((D, D), _full),
            pl.BlockSpec((D, D), _full),
            pl.BlockSpec((1, D), _full),
        ],
        out_specs=[
            pl.BlockSpec((BR, D), _row_block),
            pl.BlockSpec((BR, D), _row_block),
            pl.BlockSpec((BR, D), _row_block),
            pl.BlockSpec((BR, D), _row_block),
        ],
        out_shape=[
            jax.ShapeDtypeStruct((N, D), jnp.float32),
            jax.ShapeDtypeStruct((N, D), jnp.float32),
            jax.ShapeDtypeStruct((N, D), jnp.float32),
            jax.ShapeDtypeStruct((N, D), jnp.float32),
        ],
    )(A, dhw, d, la, wd_p, wmlp_p, bmlp_p)

    outp = pl.pallas_call(
        _layer3_kernel,
        grid=(NBLK,),
        in_specs=[
            _adj_spec(),
            pl.BlockSpec((N, D), _full),
            pl.BlockSpec((BR, 1), _row_block),
            pl.BlockSpec((1, 1), _full),
        ],
        out_specs=pl.BlockSpec((BR, D), _row_block),
        out_shape=jax.ShapeDtypeStruct((N, D), jnp.float32),
    )(A, dz, d, la)

    C = Wd.shape[1]
    logits = outp[None, :, :C]
    logits_node = ln[None, :, :C]
    return (logits, logits_node, e2, hh2)


# speedup vs baseline: 1.2554x; 1.0817x over previous
import jax
import jax.numpy as jnp
from jax.experimental import pallas as pl

N = 4096
D = 128
BR = 256
NBLK = N // BR


def _deg_proj_kernel(adj_ref, x_ref, wenc_ref, la_ref, abf_ref, d_ref, dxw_ref):
    la = la_ref[0, 0]
    a = adj_ref[...]
    abf_ref[...] = a.astype(jnp.bfloat16)
    deg = jnp.sum(a, axis=1, keepdims=True) + la
    d = jnp.where(deg > 0, jax.lax.rsqrt(deg), 0.0)
    d_ref[...] = d
    xw = jnp.dot(x_ref[...], wenc_ref[...], preferred_element_type=jnp.float32)
    dxw_ref[...] = (d * xw).astype(jnp.bfloat16)


def _spmm(adj_ref, v_ref, la, i):
    acc = jnp.dot(adj_ref[...], v_ref[...], preferred_element_type=jnp.float32)
    v_rows = v_ref[pl.ds(i * BR, BR), :].astype(jnp.float32)
    return acc + la * v_rows


def _layer1_kernel(adj_ref, v_ref, d_ref, la_ref, w2_ref, o_ref):
    i = pl.program_id(0)
    acc = _spmm(adj_ref, v_ref, la_ref[0, 0], i)
    h = jnp.maximum(d_ref[...] * acc, 0.0)
    o_ref[...] = (
        d_ref[...]
        * jnp.dot(h, w2_ref[...], preferred_element_type=jnp.float32)
    ).astype(jnp.bfloat16)


def _layer2_kernel(
    adj_ref, v_ref, d_ref, la_ref, wd_ref, wmlp_ref, bmlp_ref,
    e2_ref, hh2_ref, ln_ref, dz_ref,
):
    i = pl.program_id(0)
    acc = _spmm(adj_ref, v_ref, la_ref[0, 0], i)
    y = d_ref[...] * acc
    e2 = y[:, :D]
    hh2 = y[:, D:]
    e2_ref[...] = e2
    hh2_ref[...] = hh2
    ln_ref[...] = (
        jnp.dot(e2, wmlp_ref[...], preferred_element_type=jnp.float32)
        + bmlp_ref[...]
    )
    dz_ref[...] = (
        d_ref[...]
        * jnp.dot(hh2, wd_ref[...], preferred_element_type=jnp.float32)
    ).astype(jnp.bfloat16)


def _layer3_kernel(adj_ref, v_ref, d_ref, la_ref, o_ref):
    i = pl.program_id(0)
    acc = _spmm(adj_ref, v_ref, la_ref[0, 0], i)
    o_ref[...] = d_ref[...] * acc


def _row_block(i):
    return (i, 0)


def _full(i):
    return (0, 0)


def _adj_spec():
    return pl.BlockSpec((BR, N), _row_block)


@jax.jit
def kernel(x, adj, loop_att, We1, We2, Wh1, Wh2, Wd, Wmlp, bmlp):
    xs = x[0]
    A = adj[0]
    la = jnp.reshape(loop_att, (1, 1))

    wenc = jnp.concatenate([We1, Wh1], axis=1)
    zero = jnp.zeros((D, D), jnp.float32)
    w2 = jnp.concatenate(
        [
            jnp.concatenate([We2, zero], axis=1),
            jnp.concatenate([zero, Wh2], axis=1),
        ],
        axis=0,
    )
    wd_p = jnp.zeros((D, D), jnp.float32).at[:, : Wd.shape[1]].set(Wd)
    wmlp_p = jnp.zeros((D, D), jnp.float32).at[:, : Wmlp.shape[1]].set(Wmlp)
    bmlp_p = jnp.zeros((1, D), jnp.float32).at[0, : bmlp.shape[0]].set(bmlp)

    abf, d, dxw = pl.pallas_call(
        _deg_proj_kernel,
        grid=(NBLK,),
        in_specs=[
            _adj_spec(),
            pl.BlockSpec((BR, D), _row_block),
            pl.BlockSpec((D, 2 * D), _full),
            pl.BlockSpec((1, 1), _full),
        ],
        out_specs=[
            _adj_spec(),
            pl.BlockSpec((BR, 1), _row_block),
            pl.BlockSpec((BR, 2 * D), _row_block),
        ],
        out_shape=[
            jax.ShapeDtypeStruct((N, N), jnp.bfloat16),
            jax.ShapeDtypeStruct((N, 1), jnp.float32),
            jax.ShapeDtypeStruct((N, 2 * D), jnp.bfloat16),
        ],
    )(A, xs, wenc, la)

    dhw = pl.pallas_call(
        _layer1_kernel,
        grid=(NBLK,),
        in_specs=[
            _adj_spec(),
            pl.BlockSpec((N, 2 * D), _full),
            pl.BlockSpec((BR, 1), _row_block),
            pl.BlockSpec((1, 1), _full),
            pl.BlockSpec((2 * D, 2 * D), _full),
        ],
        out_specs=pl.BlockSpec((BR, 2 * D), _row_block),
        out_shape=jax.ShapeDtypeStruct((N, 2 * D), jnp.bfloat16),
    )(abf, dxw, d, la, w2)

    e2, hh2, ln, dz = pl.pallas_call(
        _layer2_kernel,
        grid=(NBLK,),
        in_specs=[
            _adj_spec(),
            pl.BlockSpec((N, 2 * D), _full),
            pl.BlockSpec((BR, 1), _row_block),
            pl.BlockSpec((1, 1), _full),
            pl.BlockSpec((D, D), _full),
            pl.BlockSpec((D, D), _full),
            pl.BlockSpec((1, D), _full),
        ],
        out_specs=[
            pl.BlockSpec((BR, D), _row_block),
            pl.BlockSpec((BR, D), _row_block),
            pl.BlockSpec((BR, D), _row_block),
            pl.BlockSpec((BR, D), _row_block),
        ],
        out_shape=[
            jax.ShapeDtypeStruct((N, D), jnp.float32),
            jax.ShapeDtypeStruct((N, D), jnp.float32),
            jax.ShapeDtypeStruct((N, D), jnp.float32),
            jax.ShapeDtypeStruct((N, D), jnp.bfloat16),
        ],
    )(abf, dhw, d, la, wd_p, wmlp_p, bmlp_p)

    outp = pl.pallas_call(
        _layer3_kernel,
        grid=(NBLK,),
        in_specs=[
            _adj_spec(),
            pl.BlockSpec((N, D), _full),
            pl.BlockSpec((BR, 1), _row_block),
            pl.BlockSpec((1, 1), _full),
        ],
        out_specs=pl.BlockSpec((BR, D), _row_block),
        out_shape=jax.ShapeDtypeStruct((N, D), jnp.float32),
    )(abf, dz, d, la)

    C = Wd.shape[1]
    logits = outp[None, :, :C]
    logits_node = ln[None, :, :C]
    return (logits, logits_node, e2, hh2)


# speedup vs baseline: 1.7358x; 1.3827x over previous
import jax
import jax.numpy as jnp
from jax.experimental import pallas as pl
from jax.experimental.pallas import tpu as pltpu

N = 4096
D = 128
BR = 256
NBLK = N // BR


def _kernel(
    adj_ref, x_ref, wenc_ref, w2_ref, wd_ref, wmlp_ref, bmlp_ref, la_ref,
    e2_ref, hh2_ref, ln_ref, out_ref,
    abf_ref, d_ref, va_ref, vb_ref, dz_ref,
):
    p = pl.program_id(0)
    i = pl.program_id(1)
    la = la_ref[0, 0]
    rows = pl.ds(i * BR, BR)

    @pl.when(p == 0)
    def _phase0():
        a = adj_ref[...]
        abf_ref[rows, :] = a.astype(jnp.bfloat16)
        deg = jnp.sum(a, axis=1, keepdims=True) + la
        d = jnp.where(deg > 0, jax.lax.rsqrt(deg), 0.0)
        d_ref[rows, :] = d
        xw = jnp.dot(
            x_ref[...], wenc_ref[...], preferred_element_type=jnp.float32
        )
        va_ref[rows, :] = (d * xw).astype(jnp.bfloat16)

    def _spmm(v):
        acc = jnp.dot(
            abf_ref[rows, :], v[...], preferred_element_type=jnp.float32
        )
        return acc + la * v[rows, :].astype(jnp.float32)

    @pl.when(p == 1)
    def _phase1():
        d = d_ref[rows, :]
        h = jnp.maximum(d * _spmm(va_ref), 0.0)
        hw = jnp.dot(h, w2_ref[...], preferred_element_type=jnp.float32)
        vb_ref[rows, :] = (d * hw).astype(jnp.bfloat16)

    @pl.when(p == 2)
    def _phase2():
        d = d_ref[rows, :]
        y = d * _spmm(vb_ref)
        e2 = y[:, :D]
        hh2 = y[:, D:]
        e2_ref[...] = e2
        hh2_ref[...] = hh2
        ln_ref[...] = (
            jnp.dot(e2, wmlp_ref[...], preferred_element_type=jnp.float32)
            + bmlp_ref[...]
        )
        dz_ref[rows, :] = (
            d * jnp.dot(hh2, wd_ref[...], preferred_element_type=jnp.float32)
        ).astype(jnp.bfloat16)

    @pl.when(p == 3)
    def _phase3():
        acc = jnp.dot(
            abf_ref[rows, :], dz_ref[...], preferred_element_type=jnp.float32
        )
        acc = acc + la * dz_ref[rows, :].astype(jnp.float32)
        out_ref[...] = d_ref[rows, :] * acc


def _phase0_rows(p, i):
    return (jnp.where(p == 0, i, NBLK - 1), 0)


def _phase2_rows(p, i):
    return (jnp.where(p == 2, i, jnp.where(p < 2, 0, NBLK - 1)), 0)


def _phase3_rows(p, i):
    return (jnp.where(p == 3, i, 0), 0)


def _const(p, i):
    return (0, 0)


@jax.jit
def kernel(x, adj, loop_att, We1, We2, Wh1, Wh2, Wd, Wmlp, bmlp):
    xs = x[0]
    A = adj[0]
    la = jnp.reshape(loop_att, (1, 1))

    wenc = jnp.concatenate([We1, Wh1], axis=1)
    zero = jnp.zeros((D, D), jnp.float32)
    w2 = jnp.concatenate(
        [
            jnp.concatenate([We2, zero], axis=1),
            jnp.concatenate([zero, Wh2], axis=1),
        ],
        axis=0,
    )
    wd_p = jnp.zeros((D, D), jnp.float32).at[:, : Wd.shape[1]].set(Wd)
    wmlp_p = jnp.zeros((D, D), jnp.float32).at[:, : Wmlp.shape[1]].set(Wmlp)
    bmlp_p = jnp.zeros((1, D), jnp.float32).at[0, : bmlp.shape[0]].set(bmlp)

    e2, hh2, ln, outp = pl.pallas_call(
        _kernel,
        grid=(4, NBLK),
        in_specs=[
            pl.BlockSpec((BR, N), _phase0_rows),
            pl.BlockSpec((BR, D), _phase0_rows),
            pl.BlockSpec((D, 2 * D), _const),
            pl.BlockSpec((2 * D, 2 * D), _const),
            pl.BlockSpec((D, D), _const),
            pl.BlockSpec((D, D), _const),
            pl.BlockSpec((1, D), _const),
            pl.BlockSpec((1, 1), _const),
        ],
        out_specs=[
            pl.BlockSpec((BR, D), _phase2_rows),
            pl.BlockSpec((BR, D), _phase2_rows),
            pl.BlockSpec((BR, D), _phase2_rows),
            pl.BlockSpec((BR, D), _phase3_rows),
        ],
        out_shape=[
            jax.ShapeDtypeStruct((N, D), jnp.float32),
            jax.ShapeDtypeStruct((N, D), jnp.float32),
            jax.ShapeDtypeStruct((N, D), jnp.float32),
            jax.ShapeDtypeStruct((N, D), jnp.float32),
        ],
        scratch_shapes=[
            pltpu.VMEM((N, N), jnp.bfloat16),
            pltpu.VMEM((N, 1), jnp.float32),
            pltpu.VMEM((N, 2 * D), jnp.bfloat16),
            pltpu.VMEM((N, 2 * D), jnp.bfloat16),
            pltpu.VMEM((N, D), jnp.bfloat16),
        ],
        compiler_params=pltpu.CompilerParams(
            vmem_limit_bytes=100 * 1024 * 1024,
        ),
    )(A, xs, wenc, w2, wd_p, wmlp_p, bmlp_p, la)

    C = Wd.shape[1]
    logits = outp[None, :, :C]
    logits_node = ln[None, :, :C]
    return (logits, logits_node, e2, hh2)


# speedup vs baseline: 2.0221x; 1.1649x over previous
import jax
import jax.numpy as jnp
from jax.experimental import pallas as pl
from jax.experimental.pallas import tpu as pltpu

N = 4096
D = 128
BR = 512
NBLK = N // BR


def _kernel(
    adj_ref, x_ref, wenc_ref, w2_ref, wd_ref, wmlp_ref, bmlp_ref, la_ref,
    e2_ref, hh2_ref, ln_ref, out_ref,
    abf_ref, d_ref, va_ref, vb_ref, dz_ref,
):
    p = pl.program_id(0)
    i = pl.program_id(1)
    la = la_ref[0, 0]
    rows = pl.ds(i * BR, BR)

    @pl.when(p == 0)
    def _phase0():
        a = adj_ref[...]
        abf_ref[rows, :] = a.astype(jnp.bfloat16)
        deg = jnp.sum(a, axis=1, keepdims=True) + la
        d = jnp.where(deg > 0, jax.lax.rsqrt(deg), 0.0)
        d_ref[rows, :] = d
        xw = jnp.dot(
            x_ref[...], wenc_ref[...], preferred_element_type=jnp.float32
        )
        va_ref[rows, :] = (d * xw).astype(jnp.bfloat16)

    def _spmm(v):
        acc = jnp.dot(
            abf_ref[rows, :], v[...], preferred_element_type=jnp.float32
        )
        return acc + la * v[rows, :].astype(jnp.float32)

    @pl.when(p == 1)
    def _phase1():
        d = d_ref[rows, :]
        h = jnp.maximum(d * _spmm(va_ref), 0.0)
        hw = jnp.dot(h, w2_ref[...], preferred_element_type=jnp.float32)
        vb_ref[rows, :] = (d * hw).astype(jnp.bfloat16)

    @pl.when(p == 2)
    def _phase2():
        d = d_ref[rows, :]
        y = d * _spmm(vb_ref)
        e2 = y[:, :D]
        hh2 = y[:, D:]
        e2_ref[...] = e2
        hh2_ref[...] = hh2
        ln_ref[...] = (
            jnp.dot(e2, wmlp_ref[...], preferred_element_type=jnp.float32)
            + bmlp_ref[...]
        )
        dz_ref[rows, :] = (
            d * jnp.dot(hh2, wd_ref[...], preferred_element_type=jnp.float32)
        ).astype(jnp.bfloat16)

    @pl.when(p == 3)
    def _phase3():
        acc = jnp.dot(
            abf_ref[rows, :], dz_ref[...], preferred_element_type=jnp.float32
        )
        acc = acc + la * dz_ref[rows, :].astype(jnp.float32)
        out_ref[...] = d_ref[rows, :] * acc


def _phase0_rows(p, i):
    return (jnp.where(p == 0, i, NBLK - 1), 0)


def _phase2_rows(p, i):
    return (jnp.where(p == 2, i, jnp.where(p < 2, 0, NBLK - 1)), 0)


def _phase3_rows(p, i):
    return (jnp.where(p == 3, i, 0), 0)


def _const(p, i):
    return (0, 0)


@jax.jit
def kernel(x, adj, loop_att, We1, We2, Wh1, Wh2, Wd, Wmlp, bmlp):
    xs = x[0]
    A = adj[0]
    la = jnp.reshape(loop_att, (1, 1))

    wenc = jnp.concatenate([We1, Wh1], axis=1)
    zero = jnp.zeros((D, D), jnp.float32)
    w2 = jnp.concatenate(
        [
            jnp.concatenate([We2, zero], axis=1),
            jnp.concatenate([zero, Wh2], axis=1),
        ],
        axis=0,
    )
    wd_p = jnp.zeros((D, D), jnp.float32).at[:, : Wd.shape[1]].set(Wd)
    wmlp_p = jnp.zeros((D, D), jnp.float32).at[:, : Wmlp.shape[1]].set(Wmlp)
    bmlp_p = jnp.zeros((1, D), jnp.float32).at[0, : bmlp.shape[0]].set(bmlp)

    e2, hh2, ln, outp = pl.pallas_call(
        _kernel,
        grid=(4, NBLK),
        in_specs=[
            pl.BlockSpec((BR, N), _phase0_rows),
            pl.BlockSpec((BR, D), _phase0_rows),
            pl.BlockSpec((D, 2 * D), _const),
            pl.BlockSpec((2 * D, 2 * D), _const),
            pl.BlockSpec((D, D), _const),
            pl.BlockSpec((D, D), _const),
            pl.BlockSpec((1, D), _const),
            pl.BlockSpec((1, 1), _const),
        ],
        out_specs=[
            pl.BlockSpec((BR, D), _phase2_rows),
            pl.BlockSpec((BR, D), _phase2_rows),
            pl.BlockSpec((BR, D), _phase2_rows),
            pl.BlockSpec((BR, D), _phase3_rows),
        ],
        out_shape=[
            jax.ShapeDtypeStruct((N, D), jnp.float32),
            jax.ShapeDtypeStruct((N, D), jnp.float32),
            jax.ShapeDtypeStruct((N, D), jnp.float32),
            jax.ShapeDtypeStruct((N, D), jnp.float32),
        ],
        scratch_shapes=[
            pltpu.VMEM((N, N), jnp.bfloat16),
            pltpu.VMEM((N, 1), jnp.float32),
            pltpu.VMEM((N, 2 * D), jnp.bfloat16),
            pltpu.VMEM((N, 2 * D), jnp.bfloat16),
            pltpu.VMEM((N, D), jnp.bfloat16),
        ],
        compiler_params=pltpu.CompilerParams(
            vmem_limit_bytes=100 * 1024 * 1024,
        ),
    )(A, xs, wenc, w2, wd_p, wmlp_p, bmlp_p, la)

    C = Wd.shape[1]
    logits = outp[None, :, :C]
    logits_node = ln[None, :, :C]
    return (logits, logits_node, e2, hh2)


# speedup vs baseline: 2.1995x; 1.0878x over previous
import jax
import jax.numpy as jnp
from jax.experimental import pallas as pl
from jax.experimental.pallas import tpu as pltpu

N = 4096
D = 128
BR = 512
NBLK = N // BR


def _kernel(
    adj_ref, x_ref, we1_ref, we2_ref, wh1_ref, wh2_ref, wd_ref, wmlp_ref,
    bmlp_ref, la_ref,
    e2_ref, hh2_ref, ln_ref, out_ref,
    abf_ref, d_ref, va_ref, vb_ref, dz_ref,
):
    p = pl.program_id(0)
    i = pl.program_id(1)
    la = la_ref[0, 0]
    rows = pl.ds(i * BR, BR)

    @pl.when(p == 0)
    def _phase0():
        a = adj_ref[...]
        abf_ref[rows, :] = a.astype(jnp.bfloat16)
        deg = jnp.sum(a, axis=1, keepdims=True) + la
        d = jnp.where(deg > 0, jax.lax.rsqrt(deg), 0.0)
        d_ref[rows, :] = d
        xb = x_ref[...]
        xw_e = jnp.dot(xb, we1_ref[...], preferred_element_type=jnp.float32)
        xw_h = jnp.dot(xb, wh1_ref[...], preferred_element_type=jnp.float32)
        va_ref[rows, : D] = (d * xw_e).astype(jnp.bfloat16)
        va_ref[rows, D:] = (d * xw_h).astype(jnp.bfloat16)

    def _spmm(v):
        acc = jnp.dot(
            abf_ref[rows, :], v[...], preferred_element_type=jnp.float32
        )
        return acc + la * v[rows, :].astype(jnp.float32)

    @pl.when(p == 1)
    def _phase1():
        d = d_ref[rows, :]
        h = jnp.maximum(d * _spmm(va_ref), 0.0).astype(jnp.bfloat16)
        hw_e = jnp.dot(
            h[:, : D],
            we2_ref[...].astype(jnp.bfloat16),
            preferred_element_type=jnp.float32,
        )
        hw_h = jnp.dot(
            h[:, D:],
            wh2_ref[...].astype(jnp.bfloat16),
            preferred_element_type=jnp.float32,
        )
        vb_ref[rows, : D] = (d * hw_e).astype(jnp.bfloat16)
        vb_ref[rows, D:] = (d * hw_h).astype(jnp.bfloat16)

    @pl.when(p == 2)
    def _phase2():
        d = d_ref[rows, :]
        y = d * _spmm(vb_ref)
        e2 = y[:, : D]
        hh2 = y[:, D:]
        e2_ref[...] = e2
        hh2_ref[...] = hh2
        ln_ref[...] = (
            jnp.dot(
                e2.astype(jnp.bfloat16),
                wmlp_ref[...].astype(jnp.bfloat16),
                preferred_element_type=jnp.float32,
            )
            + bmlp_ref[...]
        )
        dz_ref[rows, :] = (
            d
            * jnp.dot(
                hh2.astype(jnp.bfloat16),
                wd_ref[...].astype(jnp.bfloat16),
                preferred_element_type=jnp.float32,
            )
        ).astype(jnp.bfloat16)

    @pl.when(p == 3)
    def _phase3():
        acc = jnp.dot(
            abf_ref[rows, :], dz_ref[...], preferred_element_type=jnp.float32
        )
        acc = acc + la * dz_ref[rows, :].astype(jnp.float32)
        out_ref[...] = d_ref[rows, :] * acc


def _phase0_rows(p, i):
    return (jnp.where(p == 0, i, NBLK - 1), 0)


def _phase2_rows(p, i):
    return (jnp.where(p == 2, i, jnp.where(p < 2, 0, NBLK - 1)), 0)


def _phase3_rows(p, i):
    return (jnp.where(p == 3, i, 0), 0)


def _const(p, i):
    return (0, 0)


@jax.jit
def kernel(x, adj, loop_att, We1, We2, Wh1, Wh2, Wd, Wmlp, bmlp):
    xs = x[0]
    A = adj[0]
    la = jnp.reshape(loop_att, (1, 1))
    C = Wd.shape[1]

    e2, hh2, ln, outp = pl.pallas_call(
        _kernel,
        grid=(4, NBLK),
        in_specs=[
            pl.BlockSpec((BR, N), _phase0_rows),
            pl.BlockSpec((BR, D), _phase0_rows),
            pl.BlockSpec((D, D), _const),
            pl.BlockSpec((D, D), _const),
            pl.BlockSpec((D, D), _const),
            pl.BlockSpec((D, D), _const),
            pl.BlockSpec((D, C), _const),
            pl.BlockSpec((D, C), _const),
            pl.BlockSpec((1, C), _const),
            pl.BlockSpec((1, 1), _const),
        ],
        out_specs=[
            pl.BlockSpec((BR, D), _phase2_rows),
            pl.BlockSpec((BR, D), _phase2_rows),
            pl.BlockSpec((BR, C), _phase2_rows),
            pl.BlockSpec((BR, C), _phase3_rows),
        ],
        out_shape=[
            jax.ShapeDtypeStruct((N, D), jnp.float32),
            jax.ShapeDtypeStruct((N, D), jnp.float32),
            jax.ShapeDtypeStruct((N, C), jnp.float32),
            jax.ShapeDtypeStruct((N, C), jnp.float32),
        ],
        scratch_shapes=[
            pltpu.VMEM((N, N), jnp.bfloat16),
            pltpu.VMEM((N, 1), jnp.float32),
            pltpu.VMEM((N, 2 * D), jnp.bfloat16),
            pltpu.VMEM((N, 2 * D), jnp.bfloat16),
            pltpu.VMEM((N, C), jnp.bfloat16),
        ],
        compiler_params=pltpu.CompilerParams(
            vmem_limit_bytes=100 * 1024 * 1024,
        ),
    )(A, xs, We1, We2, Wh1, Wh2, Wd, Wmlp, bmlp[None, :], la)

    return (outp[None], ln[None], e2, hh2)
